# baseline (speedup 1.0000x reference)
import functools

import jax
import jax.numpy as jnp
from jax import lax
from jax.experimental import pallas as pl
from jax.experimental.pallas import tpu as pltpu
from jax.experimental.pallas import tpu_sc as plsc

N = 10000
NT = 10240
NC = 2
NS = 16
L = 16
NW = NC * NS
CH = 128
NB = 8
NTT = 2 * NT
RPS = NT // NS
RPSH = NTT // NS
BM = 1024

_MESH = plsc.VectorSubcoreMesh(
    core_axis_name="c", subcore_axis_name="s", num_cores=NC, num_subcores=NS
)
_SC_PARAMS = pltpu.CompilerParams(use_tc_tiling_on_sc=False)


@functools.cache
def _sc_hist(nrows_idx):
    nchunk = nrows_idx // NW
    ngroup = nchunk // NB

    def body(dst_hbm, z_hbm, out_hbm, di_all, ones_v, acc_sh, *sss):
        c = lax.axis_index("c")
        s = lax.axis_index("s")
        wid = c * NS + s
        for j in range(CH // L):
            ones_v[pl.ds(j * L, L)] = jnp.ones((L,), jnp.float32)
        pltpu.sync_copy(z_hbm.at[pl.ds(s * RPSH, RPSH)],
                        acc_sh.at[pl.ds(s * RPSH, RPSH)])
        pltpu.sync_copy(dst_hbm.at[pl.ds(wid * nchunk, nchunk)], di_all)
        plsc.subcore_barrier()

        for b in range(NB):
            pltpu.async_copy(ones_v, acc_sh.at[di_all.at[b]], sss[b],
                             add=True)

        def group(j, carry):
            i0 = j * NB
            for b in range(NB):
                pltpu.make_async_copy(ones_v, acc_sh.at[di_all.at[i0 + b]],
                                      sss[b]).wait()

                @pl.when(j < ngroup - 1)
                def _():
                    pltpu.async_copy(ones_v,
                                     acc_sh.at[di_all.at[i0 + NB + b]],
                                     sss[b], add=True)
            return carry

        lax.fori_loop(0, ngroup, group, 0)
        plsc.subcore_barrier()
        pltpu.sync_copy(acc_sh.at[pl.ds(s * RPSH, RPSH)],
                        out_hbm.at[pl.ds(c * NTT + s * RPSH, RPSH)])

    return pl.kernel(
        body,
        out_type=jax.ShapeDtypeStruct((NC * NTT,), jnp.float32),
        mesh=_MESH,
        scratch_types=[
            pltpu.VMEM((nchunk, CH), jnp.int32),
            pltpu.VMEM((CH,), jnp.float32),
            pltpu.VMEM_SHARED((NTT,), jnp.float32),
        ] + [pltpu.SemaphoreType.DMA] * NB,
        compiler_params=_SC_PARAMS,
    )


@functools.cache
def _sc_agg(f, nrows_idx):
    nchunk = nrows_idx // NW
    ngroup = nchunk // NB

    def body(g_hbm, src_hbm, dst_hbm, z_hbm, out_hbm,
             si_all, di_all, *rest):
        rs = rest[:NB]
        acc_sh = rest[NB]
        sems = rest[NB + 1:]
        sgs, sss = sems[:NB], sems[NB:]
        c = lax.axis_index("c")
        s = lax.axis_index("s")
        wid = c * NS + s
        pltpu.sync_copy(z_hbm.at[pl.ds(s * RPS, RPS)],
                        acc_sh.at[pl.ds(s * RPS, RPS)])
        pltpu.sync_copy(src_hbm.at[pl.ds(wid * nchunk, nchunk)], si_all)
        pltpu.sync_copy(dst_hbm.at[pl.ds(wid * nchunk, nchunk)], di_all)
        plsc.subcore_barrier()

        for b in range(NB):
            pltpu.async_copy(g_hbm.at[si_all.at[b]], rs[b], sgs[b])

        def group(j, carry):
            i0 = j * NB
            for b in range(NB):
                pltpu.make_async_copy(g_hbm.at[si_all.at[i0 + b]],
                                      rs[b], sgs[b]).wait()
                pltpu.async_copy(rs[b], acc_sh.at[di_all.at[i0 + b]],
                                 sss[b], add=True)

            @pl.when(j < ngroup - 1)
            def _():
                for b in range(NB):
                    pltpu.make_async_copy(rs[b], acc_sh.at[di_all.at[i0 + b]],
                                          sss[b]).wait()
                    pltpu.async_copy(g_hbm.at[si_all.at[i0 + NB + b]],
                                     rs[b], sgs[b])
            return carry

        lax.fori_loop(0, ngroup, group, 0)
        for b in range(NB):
            pltpu.make_async_copy(rs[b], acc_sh.at[di_all.at[nchunk - NB + b]],
                                  sss[b]).wait()
        plsc.subcore_barrier()
        pltpu.sync_copy(acc_sh.at[pl.ds(s * RPS, RPS)],
                        out_hbm.at[pl.ds(c * NT + s * RPS, RPS)])

    return pl.kernel(
        body,
        out_type=jax.ShapeDtypeStruct((NC * NT, f), jnp.float32),
        mesh=_MESH,
        scratch_types=[
            pltpu.VMEM((nchunk, CH), jnp.int32),
            pltpu.VMEM((nchunk, CH), jnp.int32),
        ] + [pltpu.VMEM((CH, f), jnp.float32)] * NB + [
            pltpu.VMEM_SHARED((NT, f), jnp.float32),
        ] + [pltpu.SemaphoreType.DMA] * (2 * NB),
        compiler_params=_SC_PARAMS,
    )


def _sc_hist_call(dst2d, znt):
    return _sc_hist(dst2d.shape[0])(dst2d, znt)


def _sc_agg_call(g, src2d, dst2d, zf):
    return _sc_agg(g.shape[1], src2d.shape[0])(g, src2d, dst2d, zf)


def _prep_body(x_ref, h0_ref, h1_ref, w_ref, dinv_ref, g_ref):
    deg = h0_ref[...] + h1_ref[...] + 1.0
    dinv = lax.rsqrt(deg)
    g = jnp.dot(x_ref[...], w_ref[...], preferred_element_type=jnp.float32)
    g_ref[...] = g * dinv
    dinv_ref[...] = dinv


@functools.cache
def _tc_prep(d, f, k):
    grid = NT // BM
    gb = NT // BM
    return pl.pallas_call(
        _prep_body,
        grid=(grid,),
        in_specs=[
            pl.BlockSpec((BM, d), lambda i: (i, 0)),
            pl.BlockSpec((BM, 1), lambda i: (i + k * gb, 0)),
            pl.BlockSpec((BM, 1), lambda i: (i + (2 + k) * gb, 0)),
            pl.BlockSpec((d, f), lambda i: (0, 0)),
        ],
        out_specs=[
            pl.BlockSpec((BM, 1), lambda i: (i, 0)),
            pl.BlockSpec((BM, f), lambda i: (i, 0)),
        ],
        out_shape=[
            jax.ShapeDtypeStruct((NT, 1), jnp.float32),
            jax.ShapeDtypeStruct((NT, f), jnp.float32),
        ],
    )


def _mid_body(p0_ref, p1_ref, g_ref, dinv_ref, b_ref, w_ref, out_ref):
    dinv = dinv_ref[...]
    x = jax.nn.relu(dinv * (p0_ref[...] + p1_ref[...] + g_ref[...])
                    + b_ref[...])
    i = pl.program_id(0)
    rid = i * BM + lax.broadcasted_iota(jnp.int32, (BM, 1), 0)
    x = jnp.where(rid < N, x, 0.0)
    out_ref[...] = jnp.dot(x, w_ref[...],
                           preferred_element_type=jnp.float32) * dinv


@functools.cache
def _tc_mid(fin, fout):
    grid = NT // BM
    return pl.pallas_call(
        _mid_body,
        grid=(grid,),
        in_specs=[
            pl.BlockSpec((BM, fin), lambda i: (i, 0)),
            pl.BlockSpec((BM, fin), lambda i: (i + NT // BM, 0)),
            pl.BlockSpec((BM, fin), lambda i: (i, 0)),
            pl.BlockSpec((BM, 1), lambda i: (i, 0)),
            pl.BlockSpec((1, fin), lambda i: (0, 0)),
            pl.BlockSpec((fin, fout), lambda i: (0, 0)),
        ],
        out_specs=pl.BlockSpec((BM, fout), lambda i: (i, 0)),
        out_shape=jax.ShapeDtypeStruct((NT, fout), jnp.float32),
    )


def _final_body(p0_ref, p1_ref, g_ref, dinv_ref, b_ref, out_ref):
    a = dinv_ref[...] * (p0_ref[...] + p1_ref[...] + g_ref[...]) + b_ref[...]
    i = pl.program_id(0)
    rid = i * BM + lax.broadcasted_iota(jnp.int32, (BM, 1), 0)
    out_ref[...] = jnp.where(rid < N, a, 0.0)


@functools.cache
def _tc_final(f):
    grid = NT // BM
    return pl.pallas_call(
        _final_body,
        grid=(grid,),
        in_specs=[
            pl.BlockSpec((BM, f), lambda i: (i, 0)),
            pl.BlockSpec((BM, f), lambda i: (i + NT // BM, 0)),
            pl.BlockSpec((BM, f), lambda i: (i, 0)),
            pl.BlockSpec((BM, 1), lambda i: (i, 0)),
            pl.BlockSpec((1, f), lambda i: (0, 0)),
        ],
        out_specs=pl.BlockSpec((BM, f), lambda i: (i, 0)),
        out_shape=jax.ShapeDtypeStruct((NT, f), jnp.float32),
    )


def _match_body(a1_ref, a2_ref, wm1_ref, wm2_ref, f1_ref, f2_ref):
    a1 = a1_ref[...]
    a2 = a2_ref[...]
    m1 = jnp.tanh(jnp.dot(jnp.sum(a1, axis=0, keepdims=True) * (1.0 / N),
                          wm1_ref[...], preferred_element_type=jnp.float32))
    m2 = jnp.tanh(jnp.dot(jnp.sum(a2, axis=0, keepdims=True) * (1.0 / N),
                          wm2_ref[...], preferred_element_type=jnp.float32))
    f1_ref[...] = a1 - m2
    f2_ref[...] = a2 - m1


@functools.cache
def _tc_match(f):
    return pl.pallas_call(
        _match_body,
        out_shape=[
            jax.ShapeDtypeStruct((NT, f), jnp.float32),
            jax.ShapeDtypeStruct((NT, f), jnp.float32),
        ],
    )


def _scores_body(f1_ref, f2_ref, o_ref):
    prod = lax.dot_general(f1_ref[...], f2_ref[...],
                           (((1,), (1,)), ((), ())),
                           preferred_element_type=jnp.float32)
    o_ref[...] = jax.nn.sigmoid(prod)


@functools.cache
def _tc_scores(f):
    gm = pl.cdiv(N, BM)
    return pl.pallas_call(
        _scores_body,
        grid=(gm, gm),
        in_specs=[
            pl.BlockSpec((BM, f), lambda i, j: (i, 0)),
            pl.BlockSpec((BM, f), lambda i, j: (j, 0)),
        ],
        out_specs=pl.BlockSpec((BM, BM), lambda i, j: (i, j)),
        out_shape=jax.ShapeDtypeStruct((N, N), jnp.float32),
    )


def _conv_stack(xp, src2d, dst2d, hist, k, W1, b1, W2, b2, W3, b3, zs):
    dinv, g1 = _tc_prep(xp.shape[1], W1.shape[1], k)(xp, hist, hist, W1)
    p1 = _sc_agg_call(g1, src2d, dst2d, zs[W1.shape[1]])
    g2 = _tc_mid(W1.shape[1], W2.shape[1])(p1, p1, g1, dinv, b1, W2)
    p2 = _sc_agg_call(g2, src2d, dst2d, zs[W2.shape[1]])
    g3 = _tc_mid(W2.shape[1], W3.shape[1])(p2, p2, g2, dinv, b2, W3)
    p3 = _sc_agg_call(g3, src2d, dst2d, zs[W3.shape[1]])
    return _tc_final(W3.shape[1])(p3, p3, g3, dinv, b3)


def kernel(features_1, features_2, edge_index_1, edge_index_2,
           W1, b1, W2, b2, W3, b3, Wm1, Wm2):
    n, d = features_1.shape
    e = edge_index_1.shape[1]
    epad = ((e + NW * CH - 1) // (NW * CH)) * (NW * CH)

    def pad_edges(ei):
        pad = jnp.full((2, epad - e), n, jnp.int32)
        full = jnp.concatenate([ei.astype(jnp.int32), pad], axis=1)
        return full.reshape(2, epad // CH, CH)

    ei1 = pad_edges(edge_index_1)
    ei2 = pad_edges(edge_index_2)
    xp1 = jnp.pad(features_1, ((0, NT - n), (0, 0)))
    xp2 = jnp.pad(features_2, ((0, NT - n), (0, 0)))
    b1r = b1.reshape(1, -1)
    b2r = b2.reshape(1, -1)
    b3r = b3.reshape(1, -1)
    znt = jnp.zeros((NTT,), jnp.float32)
    zs = {w.shape[1]: jnp.zeros((NT, w.shape[1]), jnp.float32)
          for w in (W1, W2, W3)}

    dsth = jnp.concatenate([ei1[1], ei2[1] + NT], axis=0)
    hist = _sc_hist_call(dsth, znt).reshape(NC * NTT, 1)
    a1 = _conv_stack(xp1, ei1[0], ei1[1], hist, 0,
                     W1, b1r, W2, b2r, W3, b3r, zs)
    a2 = _conv_stack(xp2, ei2[0], ei2[1], hist, 1,
                     W1, b1r, W2, b2r, W3, b3r, zs)
    f1e, f2e = _tc_match(W3.shape[1])(a1, a2, Wm1, Wm2)
    return _tc_scores(W3.shape[1])(f1e, f2e)


# speedup vs baseline: 1.0689x; 1.0689x over previous
import functools

import jax
import jax.numpy as jnp
from jax import lax
from jax.experimental import pallas as pl
from jax.experimental.pallas import tpu as pltpu
from jax.experimental.pallas import tpu_sc as plsc

N = 10000
NT = 10240
NC = 2
NS = 16
L = 16
NW = NC * NS
CH = 128
NB = 8
RPS = NT // NS
BM = 1024

_MESH = plsc.VectorSubcoreMesh(
    core_axis_name="c", subcore_axis_name="s", num_cores=NC, num_subcores=NS
)
_SC_PARAMS = pltpu.CompilerParams(use_tc_tiling_on_sc=False)


@functools.cache
def _sc_hist(nrows_idx):
    nchunk = nrows_idx // NW
    ngroup = nchunk // NB

    def body(dst_hbm, z_hbm, out_hbm, di_all, ones_v, acc_sh, *sss):
        c = lax.axis_index("c")
        s = lax.axis_index("s")
        wid = c * NS + s
        for j in range(CH // L):
            ones_v[pl.ds(j * L, L)] = jnp.ones((L,), jnp.float32)
        pltpu.sync_copy(z_hbm.at[pl.ds(s * RPS, RPS)],
                        acc_sh.at[pl.ds(s * RPS, RPS)])
        pltpu.sync_copy(dst_hbm.at[pl.ds(wid * nchunk, nchunk)], di_all)
        plsc.subcore_barrier()

        for b in range(NB):
            pltpu.async_copy(ones_v, acc_sh.at[di_all.at[b]], sss[b],
                             add=True)

        def group(j, carry):
            i0 = j * NB
            for b in range(NB):
                pltpu.make_async_copy(ones_v, acc_sh.at[di_all.at[i0 + b]],
                                      sss[b]).wait()

                @pl.when(j < ngroup - 1)
                def _():
                    pltpu.async_copy(ones_v,
                                     acc_sh.at[di_all.at[i0 + NB + b]],
                                     sss[b], add=True)
            return carry

        lax.fori_loop(0, ngroup, group, 0)
        plsc.subcore_barrier()
        pltpu.sync_copy(acc_sh.at[pl.ds(s * RPS, RPS)],
                        out_hbm.at[pl.ds(c * NT + s * RPS, RPS)])

    return pl.kernel(
        body,
        out_type=jax.ShapeDtypeStruct((NC * NT,), jnp.float32),
        mesh=_MESH,
        scratch_types=[
            pltpu.VMEM((nchunk, CH), jnp.int32),
            pltpu.VMEM((CH,), jnp.float32),
            pltpu.VMEM_SHARED((NT,), jnp.float32),
        ] + [pltpu.SemaphoreType.DMA] * NB,
        compiler_params=_SC_PARAMS,
    )


@functools.cache
def _sc_agg(f, nrows_idx):
    nchunk = nrows_idx // NW
    ngroup = nchunk // NB

    def body(g_hbm, src_hbm, dst_hbm, z_hbm, out_hbm,
             si_all, di_all, *rest):
        rs = rest[:NB]
        acc_sh = rest[NB]
        sems = rest[NB + 1:]
        sgs, sss = sems[:NB], sems[NB:]
        c = lax.axis_index("c")
        s = lax.axis_index("s")
        wid = c * NS + s
        pltpu.sync_copy(z_hbm.at[pl.ds(s * RPS, RPS)],
                        acc_sh.at[pl.ds(s * RPS, RPS)])
        pltpu.sync_copy(src_hbm.at[pl.ds(wid * nchunk, nchunk)], si_all)
        pltpu.sync_copy(dst_hbm.at[pl.ds(wid * nchunk, nchunk)], di_all)
        plsc.subcore_barrier()

        for b in range(NB):
            pltpu.async_copy(g_hbm.at[si_all.at[b]], rs[b], sgs[b])

        def group(j, carry):
            i0 = j * NB
            for b in range(NB):
                pltpu.make_async_copy(g_hbm.at[si_all.at[i0 + b]],
                                      rs[b], sgs[b]).wait()
                pltpu.async_copy(rs[b], acc_sh.at[di_all.at[i0 + b]],
                                 sss[b], add=True)

            @pl.when(j < ngroup - 1)
            def _():
                for b in range(NB):
                    pltpu.make_async_copy(rs[b], acc_sh.at[di_all.at[i0 + b]],
                                          sss[b]).wait()
                    pltpu.async_copy(g_hbm.at[si_all.at[i0 + NB + b]],
                                     rs[b], sgs[b])
            return carry

        lax.fori_loop(0, ngroup, group, 0)
        for b in range(NB):
            pltpu.make_async_copy(rs[b], acc_sh.at[di_all.at[nchunk - NB + b]],
                                  sss[b]).wait()
        plsc.subcore_barrier()
        pltpu.sync_copy(acc_sh.at[pl.ds(s * RPS, RPS)],
                        out_hbm.at[pl.ds(c * NT + s * RPS, RPS)])

    return pl.kernel(
        body,
        out_type=jax.ShapeDtypeStruct((NC * NT, f), jnp.float32),
        mesh=_MESH,
        scratch_types=[
            pltpu.VMEM((nchunk, CH), jnp.int32),
            pltpu.VMEM((nchunk, CH), jnp.int32),
        ] + [pltpu.VMEM((CH, f), jnp.float32)] * NB + [
            pltpu.VMEM_SHARED((NT, f), jnp.float32),
        ] + [pltpu.SemaphoreType.DMA] * (2 * NB),
        compiler_params=_SC_PARAMS,
    )


def _sc_hist_call(dst2d, znt):
    return _sc_hist(dst2d.shape[0])(dst2d, znt)


def _sc_agg_call(g, src2d, dst2d, zf):
    return _sc_agg(g.shape[1], src2d.shape[0])(g, src2d, dst2d, zf)


def _prep_body(x_ref, h0_ref, h1_ref, w_ref, dinv_ref, g_ref):
    deg = h0_ref[...] + h1_ref[...] + 1.0
    dinv = lax.rsqrt(deg)
    g = jnp.dot(x_ref[...], w_ref[...], preferred_element_type=jnp.float32)
    g_ref[...] = g * dinv
    dinv_ref[...] = dinv


@functools.cache
def _tc_prep(d, f):
    grid = NT // BM
    return pl.pallas_call(
        _prep_body,
        grid=(grid,),
        in_specs=[
            pl.BlockSpec((BM, d), lambda i: (i, 0)),
            pl.BlockSpec((BM, 1), lambda i: (i, 0)),
            pl.BlockSpec((BM, 1), lambda i: (i + NT // BM, 0)),
            pl.BlockSpec((d, f), lambda i: (0, 0)),
        ],
        out_specs=[
            pl.BlockSpec((BM, 1), lambda i: (i, 0)),
            pl.BlockSpec((BM, f), lambda i: (i, 0)),
        ],
        out_shape=[
            jax.ShapeDtypeStruct((NT, 1), jnp.float32),
            jax.ShapeDtypeStruct((NT, f), jnp.float32),
        ],
    )


def _mid_body(p0_ref, p1_ref, g_ref, dinv_ref, b_ref, w_ref, out_ref):
    dinv = dinv_ref[...]
    x = jax.nn.relu(dinv * (p0_ref[...] + p1_ref[...] + g_ref[...])
                    + b_ref[...])
    i = pl.program_id(0)
    rid = i * BM + lax.broadcasted_iota(jnp.int32, (BM, 1), 0)
    x = jnp.where(rid < N, x, 0.0)
    out_ref[...] = jnp.dot(x, w_ref[...],
                           preferred_element_type=jnp.float32) * dinv


@functools.cache
def _tc_mid(fin, fout):
    grid = NT // BM
    return pl.pallas_call(
        _mid_body,
        grid=(grid,),
        in_specs=[
            pl.BlockSpec((BM, fin), lambda i: (i, 0)),
            pl.BlockSpec((BM, fin), lambda i: (i + NT // BM, 0)),
            pl.BlockSpec((BM, fin), lambda i: (i, 0)),
            pl.BlockSpec((BM, 1), lambda i: (i, 0)),
            pl.BlockSpec((1, fin), lambda i: (0, 0)),
            pl.BlockSpec((fin, fout), lambda i: (0, 0)),
        ],
        out_specs=pl.BlockSpec((BM, fout), lambda i: (i, 0)),
        out_shape=jax.ShapeDtypeStruct((NT, fout), jnp.float32),
    )


def _final_body(p0_ref, p1_ref, g_ref, dinv_ref, b_ref, out_ref):
    a = dinv_ref[...] * (p0_ref[...] + p1_ref[...] + g_ref[...]) + b_ref[...]
    i = pl.program_id(0)
    rid = i * BM + lax.broadcasted_iota(jnp.int32, (BM, 1), 0)
    out_ref[...] = jnp.where(rid < N, a, 0.0)


@functools.cache
def _tc_final(f):
    grid = NT // BM
    return pl.pallas_call(
        _final_body,
        grid=(grid,),
        in_specs=[
            pl.BlockSpec((BM, f), lambda i: (i, 0)),
            pl.BlockSpec((BM, f), lambda i: (i + NT // BM, 0)),
            pl.BlockSpec((BM, f), lambda i: (i, 0)),
            pl.BlockSpec((BM, 1), lambda i: (i, 0)),
            pl.BlockSpec((1, f), lambda i: (0, 0)),
        ],
        out_specs=pl.BlockSpec((BM, f), lambda i: (i, 0)),
        out_shape=jax.ShapeDtypeStruct((NT, f), jnp.float32),
    )


def _match_body(a1_ref, a2_ref, wm1_ref, wm2_ref, f1_ref, f2_ref):
    a1 = a1_ref[...]
    a2 = a2_ref[...]
    m1 = jnp.tanh(jnp.dot(jnp.sum(a1, axis=0, keepdims=True) * (1.0 / N),
                          wm1_ref[...], preferred_element_type=jnp.float32))
    m2 = jnp.tanh(jnp.dot(jnp.sum(a2, axis=0, keepdims=True) * (1.0 / N),
                          wm2_ref[...], preferred_element_type=jnp.float32))
    f1_ref[...] = a1 - m2
    f2_ref[...] = a2 - m1


@functools.cache
def _tc_match(f):
    return pl.pallas_call(
        _match_body,
        out_shape=[
            jax.ShapeDtypeStruct((NT, f), jnp.float32),
            jax.ShapeDtypeStruct((NT, f), jnp.float32),
        ],
    )


def _scores_body(f1_ref, f2_ref, o_ref):
    prod = lax.dot_general(f1_ref[...], f2_ref[...],
                           (((1,), (1,)), ((), ())),
                           preferred_element_type=jnp.float32)
    o_ref[...] = jax.nn.sigmoid(prod)


@functools.cache
def _tc_scores(f):
    gm = pl.cdiv(N, BM)
    return pl.pallas_call(
        _scores_body,
        grid=(gm, gm),
        in_specs=[
            pl.BlockSpec((BM, f), lambda i, j: (i, 0)),
            pl.BlockSpec((BM, f), lambda i, j: (j, 0)),
        ],
        out_specs=pl.BlockSpec((BM, BM), lambda i, j: (i, j)),
        out_shape=jax.ShapeDtypeStruct((N, N), jnp.float32),
    )


def _conv_stack(xp, src2d, dst2d, W1, b1, W2, b2, W3, b3, znt, zs):
    hist = _sc_hist_call(dst2d, znt).reshape(NC * NT, 1)
    dinv, g1 = _tc_prep(xp.shape[1], W1.shape[1])(xp, hist, hist, W1)
    p1 = _sc_agg_call(g1, src2d, dst2d, zs[W1.shape[1]])
    g2 = _tc_mid(W1.shape[1], W2.shape[1])(p1, p1, g1, dinv, b1, W2)
    p2 = _sc_agg_call(g2, src2d, dst2d, zs[W2.shape[1]])
    g3 = _tc_mid(W2.shape[1], W3.shape[1])(p2, p2, g2, dinv, b2, W3)
    p3 = _sc_agg_call(g3, src2d, dst2d, zs[W3.shape[1]])
    return _tc_final(W3.shape[1])(p3, p3, g3, dinv, b3)


def kernel(features_1, features_2, edge_index_1, edge_index_2,
           W1, b1, W2, b2, W3, b3, Wm1, Wm2):
    n, d = features_1.shape
    e = edge_index_1.shape[1]
    epad = ((e + NW * CH - 1) // (NW * CH)) * (NW * CH)

    def pad_edges(ei):
        pad = jnp.full((2, epad - e), n, jnp.int32)
        full = jnp.concatenate([ei.astype(jnp.int32), pad], axis=1)
        return full.reshape(2, epad // CH, CH)

    ei1 = pad_edges(edge_index_1)
    ei2 = pad_edges(edge_index_2)
    xp1 = jnp.pad(features_1, ((0, NT - n), (0, 0)))
    xp2 = jnp.pad(features_2, ((0, NT - n), (0, 0)))
    b1r = b1.reshape(1, -1)
    b2r = b2.reshape(1, -1)
    b3r = b3.reshape(1, -1)
    znt = jnp.zeros((NT,), jnp.float32)
    zs = {w.shape[1]: jnp.zeros((NT, w.shape[1]), jnp.float32)
          for w in (W1, W2, W3)}

    a1 = _conv_stack(xp1, ei1[0], ei1[1], W1, b1r, W2, b2r, W3, b3r, znt, zs)
    a2 = _conv_stack(xp2, ei2[0], ei2[1], W1, b1r, W2, b2r, W3, b3r, znt, zs)
    f1e, f2e = _tc_match(W3.shape[1])(a1, a2, Wm1, Wm2)
    return _tc_scores(W3.shape[1])(f1e, f2e)


# speedup vs baseline: 1.0842x; 1.0143x over previous
import functools

import jax
import jax.numpy as jnp
from jax import lax
from jax.experimental import pallas as pl
from jax.experimental.pallas import tpu as pltpu
from jax.experimental.pallas import tpu_sc as plsc

N = 10000
NT = 10240
NC = 2
NS = 16
L = 16
NW = NC * NS
CH = 128
NB = 8
RPS = NT // NS
BM = 1024

_MESH = plsc.VectorSubcoreMesh(
    core_axis_name="c", subcore_axis_name="s", num_cores=NC, num_subcores=NS
)
_SC_PARAMS = pltpu.CompilerParams(use_tc_tiling_on_sc=False)


@functools.cache
def _sc_hist(nrows_idx):
    nchunk = nrows_idx // NW
    ngroup = nchunk // NB

    def body(dst_hbm, z_hbm, out_hbm, di_all, ones_v, acc_sh, *sss):
        c = lax.axis_index("c")
        s = lax.axis_index("s")
        wid = c * NS + s
        for j in range(CH // L):
            ones_v[pl.ds(j * L, L)] = jnp.ones((L,), jnp.float32)
        pltpu.sync_copy(z_hbm.at[pl.ds(s * RPS, RPS)],
                        acc_sh.at[pl.ds(s * RPS, RPS)])
        pltpu.sync_copy(dst_hbm.at[pl.ds(wid * nchunk, nchunk)], di_all)
        plsc.subcore_barrier()

        for b in range(NB):
            pltpu.async_copy(ones_v, acc_sh.at[di_all.at[b]], sss[b],
                             add=True)

        def group(j, carry):
            i0 = j * NB
            for b in range(NB):
                pltpu.make_async_copy(ones_v, acc_sh.at[di_all.at[i0 + b]],
                                      sss[b]).wait()

                @pl.when(j < ngroup - 1)
                def _():
                    pltpu.async_copy(ones_v,
                                     acc_sh.at[di_all.at[i0 + NB + b]],
                                     sss[b], add=True)
            return carry

        lax.fori_loop(0, ngroup, group, 0)
        plsc.subcore_barrier()
        pltpu.sync_copy(acc_sh.at[pl.ds(s * RPS, RPS)],
                        out_hbm.at[pl.ds(c * NT + s * RPS, RPS)])

    return pl.kernel(
        body,
        out_type=jax.ShapeDtypeStruct((NC * NT,), jnp.float32),
        mesh=_MESH,
        scratch_types=[
            pltpu.VMEM((nchunk, CH), jnp.int32),
            pltpu.VMEM((CH,), jnp.float32),
            pltpu.VMEM_SHARED((NT,), jnp.float32),
        ] + [pltpu.SemaphoreType.DMA] * NB,
        compiler_params=_SC_PARAMS,
    )


@functools.cache
def _sc_agg(f, nrows_idx):
    nchunk = nrows_idx // NW
    ngroup = nchunk // NB

    def body(g_hbm, src_hbm, dst_hbm, z_hbm, out_hbm,
             si_all, di_all, *rest):
        rs = rest[:NB]
        acc_sh = rest[NB]
        sems = rest[NB + 1:]
        sgs, sss = sems[:NB], sems[NB:]
        c = lax.axis_index("c")
        s = lax.axis_index("s")
        wid = c * NS + s
        pltpu.sync_copy(src_hbm.at[pl.ds(wid * nchunk, nchunk)], si_all)
        pltpu.sync_copy(dst_hbm.at[pl.ds(wid * nchunk, nchunk)], di_all)

        for b in range(NB):
            pltpu.async_copy(g_hbm.at[si_all.at[b]], rs[b], sgs[b])

        pltpu.sync_copy(z_hbm.at[pl.ds(s * RPS, RPS)],
                        acc_sh.at[pl.ds(s * RPS, RPS)])
        plsc.subcore_barrier()

        def group(j, carry):
            i0 = j * NB
            for b in range(NB):
                pltpu.make_async_copy(g_hbm.at[si_all.at[i0 + b]],
                                      rs[b], sgs[b]).wait()
                pltpu.async_copy(rs[b], acc_sh.at[di_all.at[i0 + b]],
                                 sss[b], add=True)

            @pl.when(j < ngroup - 1)
            def _():
                for b in range(NB):
                    pltpu.make_async_copy(rs[b], acc_sh.at[di_all.at[i0 + b]],
                                          sss[b]).wait()
                    pltpu.async_copy(g_hbm.at[si_all.at[i0 + NB + b]],
                                     rs[b], sgs[b])
            return carry

        lax.fori_loop(0, ngroup, group, 0)
        for b in range(NB):
            pltpu.make_async_copy(rs[b], acc_sh.at[di_all.at[nchunk - NB + b]],
                                  sss[b]).wait()
        plsc.subcore_barrier()
        pltpu.sync_copy(acc_sh.at[pl.ds(s * RPS, RPS)],
                        out_hbm.at[pl.ds(c * NT + s * RPS, RPS)])

    return pl.kernel(
        body,
        out_type=jax.ShapeDtypeStruct((NC * NT, f), jnp.float32),
        mesh=_MESH,
        scratch_types=[
            pltpu.VMEM((nchunk, CH), jnp.int32),
            pltpu.VMEM((nchunk, CH), jnp.int32),
        ] + [pltpu.VMEM((CH, f), jnp.float32)] * NB + [
            pltpu.VMEM_SHARED((NT, f), jnp.float32),
        ] + [pltpu.SemaphoreType.DMA] * (2 * NB),
        compiler_params=_SC_PARAMS,
    )


def _sc_hist_call(dst2d, znt):
    return _sc_hist(dst2d.shape[0])(dst2d, znt)


def _sc_agg_call(g, src2d, dst2d, zf):
    return _sc_agg(g.shape[1], src2d.shape[0])(g, src2d, dst2d, zf)


def _prep_body(x_ref, h0_ref, h1_ref, w_ref, dinv_ref, g_ref):
    deg = h0_ref[...] + h1_ref[...] + 1.0
    dinv = lax.rsqrt(deg)
    g = jnp.dot(x_ref[...], w_ref[...], preferred_element_type=jnp.float32)
    g_ref[...] = g * dinv
    dinv_ref[...] = dinv


@functools.cache
def _tc_prep(d, f):
    grid = NT // BM
    return pl.pallas_call(
        _prep_body,
        grid=(grid,),
        in_specs=[
            pl.BlockSpec((BM, d), lambda i: (i, 0)),
            pl.BlockSpec((BM, 1), lambda i: (i, 0)),
            pl.BlockSpec((BM, 1), lambda i: (i + NT // BM, 0)),
            pl.BlockSpec((d, f), lambda i: (0, 0)),
        ],
        out_specs=[
            pl.BlockSpec((BM, 1), lambda i: (i, 0)),
            pl.BlockSpec((BM, f), lambda i: (i, 0)),
        ],
        out_shape=[
            jax.ShapeDtypeStruct((NT, 1), jnp.float32),
            jax.ShapeDtypeStruct((NT, f), jnp.float32),
        ],
    )


def _mid_body(p0_ref, p1_ref, g_ref, dinv_ref, b_ref, w_ref, out_ref):
    dinv = dinv_ref[...]
    x = jax.nn.relu(dinv * (p0_ref[...] + p1_ref[...] + g_ref[...])
                    + b_ref[...])
    i = pl.program_id(0)
    rid = i * BM + lax.broadcasted_iota(jnp.int32, (BM, 1), 0)
    x = jnp.where(rid < N, x, 0.0)
    out_ref[...] = jnp.dot(x, w_ref[...],
                           preferred_element_type=jnp.float32) * dinv


@functools.cache
def _tc_mid(fin, fout):
    grid = NT // BM
    return pl.pallas_call(
        _mid_body,
        grid=(grid,),
        in_specs=[
            pl.BlockSpec((BM, fin), lambda i: (i, 0)),
            pl.BlockSpec((BM, fin), lambda i: (i + NT // BM, 0)),
            pl.BlockSpec((BM, fin), lambda i: (i, 0)),
            pl.BlockSpec((BM, 1), lambda i: (i, 0)),
            pl.BlockSpec((1, fin), lambda i: (0, 0)),
            pl.BlockSpec((fin, fout), lambda i: (0, 0)),
        ],
        out_specs=pl.BlockSpec((BM, fout), lambda i: (i, 0)),
        out_shape=jax.ShapeDtypeStruct((NT, fout), jnp.float32),
    )


def _final_body(p0_ref, p1_ref, g_ref, dinv_ref, b_ref, out_ref):
    a = dinv_ref[...] * (p0_ref[...] + p1_ref[...] + g_ref[...]) + b_ref[...]
    i = pl.program_id(0)
    rid = i * BM + lax.broadcasted_iota(jnp.int32, (BM, 1), 0)
    out_ref[...] = jnp.where(rid < N, a, 0.0)


@functools.cache
def _tc_final(f):
    grid = NT // BM
    return pl.pallas_call(
        _final_body,
        grid=(grid,),
        in_specs=[
            pl.BlockSpec((BM, f), lambda i: (i, 0)),
            pl.BlockSpec((BM, f), lambda i: (i + NT // BM, 0)),
            pl.BlockSpec((BM, f), lambda i: (i, 0)),
            pl.BlockSpec((BM, 1), lambda i: (i, 0)),
            pl.BlockSpec((1, f), lambda i: (0, 0)),
        ],
        out_specs=pl.BlockSpec((BM, f), lambda i: (i, 0)),
        out_shape=jax.ShapeDtypeStruct((NT, f), jnp.float32),
    )


def _match_body(a1_ref, a2_ref, wm1_ref, wm2_ref, f1_ref, f2_ref):
    a1 = a1_ref[...]
    a2 = a2_ref[...]
    m1 = jnp.tanh(jnp.dot(jnp.sum(a1, axis=0, keepdims=True) * (1.0 / N),
                          wm1_ref[...], preferred_element_type=jnp.float32))
    m2 = jnp.tanh(jnp.dot(jnp.sum(a2, axis=0, keepdims=True) * (1.0 / N),
                          wm2_ref[...], preferred_element_type=jnp.float32))
    f1_ref[...] = a1 - m2
    f2_ref[...] = a2 - m1


@functools.cache
def _tc_match(f):
    return pl.pallas_call(
        _match_body,
        out_shape=[
            jax.ShapeDtypeStruct((NT, f), jnp.float32),
            jax.ShapeDtypeStruct((NT, f), jnp.float32),
        ],
    )


def _scores_body(f1_ref, f2_ref, o_ref):
    prod = lax.dot_general(f1_ref[...], f2_ref[...],
                           (((1,), (1,)), ((), ())),
                           preferred_element_type=jnp.float32)
    o_ref[...] = jax.nn.sigmoid(prod)


@functools.cache
def _tc_scores(f):
    gm = pl.cdiv(N, BM)
    return pl.pallas_call(
        _scores_body,
        grid=(gm, gm),
        in_specs=[
            pl.BlockSpec((BM, f), lambda i, j: (i, 0)),
            pl.BlockSpec((BM, f), lambda i, j: (j, 0)),
        ],
        out_specs=pl.BlockSpec((BM, BM), lambda i, j: (i, j)),
        out_shape=jax.ShapeDtypeStruct((N, N), jnp.float32),
    )


def _conv_stack(xp, src2d, dst2d, W1, b1, W2, b2, W3, b3, znt, zs):
    hist = _sc_hist_call(dst2d, znt).reshape(NC * NT, 1)
    dinv, g1 = _tc_prep(xp.shape[1], W1.shape[1])(xp, hist, hist, W1)
    p1 = _sc_agg_call(g1, src2d, dst2d, zs[W1.shape[1]])
    g2 = _tc_mid(W1.shape[1], W2.shape[1])(p1, p1, g1, dinv, b1, W2)
    p2 = _sc_agg_call(g2, src2d, dst2d, zs[W2.shape[1]])
    g3 = _tc_mid(W2.shape[1], W3.shape[1])(p2, p2, g2, dinv, b2, W3)
    p3 = _sc_agg_call(g3, src2d, dst2d, zs[W3.shape[1]])
    return _tc_final(W3.shape[1])(p3, p3, g3, dinv, b3)


def kernel(features_1, features_2, edge_index_1, edge_index_2,
           W1, b1, W2, b2, W3, b3, Wm1, Wm2):
    n, d = features_1.shape
    e = edge_index_1.shape[1]
    epad = ((e + NW * CH - 1) // (NW * CH)) * (NW * CH)

    def pad_edges(ei):
        pad = jnp.full((2, epad - e), n, jnp.int32)
        full = jnp.concatenate([ei.astype(jnp.int32), pad], axis=1)
        return full.reshape(2, epad // CH, CH)

    ei1 = pad_edges(edge_index_1)
    ei2 = pad_edges(edge_index_2)
    xp1 = jnp.pad(features_1, ((0, NT - n), (0, 0)))
    xp2 = jnp.pad(features_2, ((0, NT - n), (0, 0)))
    b1r = b1.reshape(1, -1)
    b2r = b2.reshape(1, -1)
    b3r = b3.reshape(1, -1)
    znt = jnp.zeros((NT,), jnp.float32)
    zs = {w.shape[1]: jnp.zeros((NT, w.shape[1]), jnp.float32)
          for w in (W1, W2, W3)}

    a1 = _conv_stack(xp1, ei1[0], ei1[1], W1, b1r, W2, b2r, W3, b3r, znt, zs)
    a2 = _conv_stack(xp2, ei2[0], ei2[1], W1, b1r, W2, b2r, W3, b3r, znt, zs)
    f1e, f2e = _tc_match(W3.shape[1])(a1, a2, Wm1, Wm2)
    return _tc_scores(W3.shape[1])(f1e, f2e)


# speedup vs baseline: 1.1224x; 1.0353x over previous
import functools

import jax
import jax.numpy as jnp
from jax import lax
from jax.experimental import pallas as pl
from jax.experimental.pallas import tpu as pltpu
from jax.experimental.pallas import tpu_sc as plsc

N = 10000
NT = 10240
NC = 2
NS = 16
L = 16
NW = NC * NS
CH = 128
NB = 8
RPS = NT // NS
BM = 1024

_MESH = plsc.VectorSubcoreMesh(
    core_axis_name="c", subcore_axis_name="s", num_cores=NC, num_subcores=NS
)
_SC_PARAMS = pltpu.CompilerParams(use_tc_tiling_on_sc=False)


@functools.cache
def _sc_hist(nrows_idx):
    nchunk = nrows_idx // NW
    ngroup = nchunk // NB

    def body(dst_hbm, z_hbm, out_hbm, di_all, ones_v, acc_sh, *sss):
        c = lax.axis_index("c")
        s = lax.axis_index("s")
        wid = c * NS + s
        for j in range(CH // L):
            ones_v[pl.ds(j * L, L)] = jnp.ones((L,), jnp.float32)
        pltpu.sync_copy(z_hbm.at[pl.ds(s * RPS, RPS)],
                        acc_sh.at[pl.ds(s * RPS, RPS)])
        pltpu.sync_copy(dst_hbm.at[pl.ds(wid * nchunk, nchunk)], di_all)
        plsc.subcore_barrier()

        for b in range(NB):
            pltpu.async_copy(ones_v, acc_sh.at[di_all.at[b]], sss[b],
                             add=True)

        def group(j, carry):
            i0 = j * NB
            for b in range(NB):
                pltpu.make_async_copy(ones_v, acc_sh.at[di_all.at[i0 + b]],
                                      sss[b]).wait()

                @pl.when(j < ngroup - 1)
                def _():
                    pltpu.async_copy(ones_v,
                                     acc_sh.at[di_all.at[i0 + NB + b]],
                                     sss[b], add=True)
            return carry

        lax.fori_loop(0, ngroup, group, 0)
        plsc.subcore_barrier()
        pltpu.sync_copy(acc_sh.at[pl.ds(s * RPS, RPS)],
                        out_hbm.at[pl.ds(c * NT + s * RPS, RPS)])

    return pl.kernel(
        body,
        out_type=jax.ShapeDtypeStruct((NC * NT,), jnp.float32),
        mesh=_MESH,
        scratch_types=[
            pltpu.VMEM((nchunk, CH), jnp.int32),
            pltpu.VMEM((CH,), jnp.float32),
            pltpu.VMEM_SHARED((NT,), jnp.float32),
        ] + [pltpu.SemaphoreType.DMA] * NB,
        compiler_params=_SC_PARAMS,
    )


@functools.cache
def _sc_agg(f, nrows_idx):
    nchunk = nrows_idx // NW
    ngroup = nchunk // NB

    def body(g_hbm, src_hbm, dst_hbm, z_hbm, out_hbm,
             si_all, di_all, *rest):
        rs = rest[:NB]
        acc_sh = rest[NB]
        sems = rest[NB + 1:]
        sgs, sss = sems[:NB], sems[NB:]
        c = lax.axis_index("c")
        s = lax.axis_index("s")
        wid = c * NS + s
        pltpu.sync_copy(src_hbm.at[pl.ds(wid * nchunk, nchunk)], si_all)
        pltpu.sync_copy(dst_hbm.at[pl.ds(wid * nchunk, nchunk)], di_all)

        for b in range(NB):
            pltpu.async_copy(g_hbm.at[si_all.at[b]], rs[b], sgs[b])

        pltpu.sync_copy(z_hbm.at[pl.ds(s * RPS, RPS)],
                        acc_sh.at[pl.ds(s * RPS, RPS)])
        plsc.subcore_barrier()

        def group(j, carry):
            i0 = j * NB
            for b in range(NB):
                pltpu.make_async_copy(g_hbm.at[si_all.at[i0 + b]],
                                      rs[b], sgs[b]).wait()
                pltpu.async_copy(rs[b], acc_sh.at[di_all.at[i0 + b]],
                                 sss[b], add=True)

            @pl.when(j < ngroup - 1)
            def _():
                for b in range(NB):
                    pltpu.make_async_copy(rs[b], acc_sh.at[di_all.at[i0 + b]],
                                          sss[b]).wait()
                    pltpu.async_copy(g_hbm.at[si_all.at[i0 + NB + b]],
                                     rs[b], sgs[b])
            return carry

        lax.fori_loop(0, ngroup, group, 0)
        for b in range(NB):
            pltpu.make_async_copy(rs[b], acc_sh.at[di_all.at[nchunk - NB + b]],
                                  sss[b]).wait()
        plsc.subcore_barrier()
        pltpu.sync_copy(acc_sh.at[pl.ds(s * RPS, RPS)],
                        out_hbm.at[pl.ds(c * NT + s * RPS, RPS)])

    return pl.kernel(
        body,
        out_type=jax.ShapeDtypeStruct((NC * NT, f), jnp.float32),
        mesh=_MESH,
        scratch_types=[
            pltpu.VMEM((nchunk, CH), jnp.int32),
            pltpu.VMEM((nchunk, CH), jnp.int32),
        ] + [pltpu.VMEM((CH, f), jnp.float32)] * NB + [
            pltpu.VMEM_SHARED((NT, f), jnp.float32),
        ] + [pltpu.SemaphoreType.DMA] * (2 * NB),
        compiler_params=_SC_PARAMS,
    )


def _sc_hist_call(dst2d, znt):
    return _sc_hist(dst2d.shape[0])(dst2d, znt)


def _sc_agg_call(g, src2d, dst2d, zf):
    return _sc_agg(g.shape[1], src2d.shape[0])(g, src2d, dst2d, zf)


def _prep_body(x_ref, h0_ref, h1_ref, w_ref, dinv_ref, g_ref):
    deg = h0_ref[...] + h1_ref[...] + 1.0
    dinv = lax.rsqrt(deg)
    g = jnp.dot(x_ref[...], w_ref[...], preferred_element_type=jnp.float32)
    g_ref[...] = g * dinv
    dinv_ref[...] = dinv


@functools.cache
def _tc_prep(d, f):
    grid = NT // BM
    return pl.pallas_call(
        _prep_body,
        grid=(grid,),
        in_specs=[
            pl.BlockSpec((BM, d), lambda i: (i, 0)),
            pl.BlockSpec((BM, 1), lambda i: (i, 0)),
            pl.BlockSpec((BM, 1), lambda i: (i + NT // BM, 0)),
            pl.BlockSpec((d, f), lambda i: (0, 0)),
        ],
        out_specs=[
            pl.BlockSpec((BM, 1), lambda i: (i, 0)),
            pl.BlockSpec((BM, f), lambda i: (i, 0)),
        ],
        out_shape=[
            jax.ShapeDtypeStruct((NT, 1), jnp.float32),
            jax.ShapeDtypeStruct((NT, f), jnp.float32),
        ],
    )


def _mid_body(p0_ref, p1_ref, g_ref, dinv_ref, b_ref, w_ref, out_ref):
    dinv = dinv_ref[...]
    x = jax.nn.relu(dinv * (p0_ref[...] + p1_ref[...] + g_ref[...])
                    + b_ref[...])
    i = pl.program_id(0)
    rid = i * BM + lax.broadcasted_iota(jnp.int32, (BM, 1), 0)
    x = jnp.where(rid < N, x, 0.0)
    out_ref[...] = jnp.dot(x, w_ref[...],
                           preferred_element_type=jnp.float32) * dinv


@functools.cache
def _tc_mid(fin, fout):
    grid = NT // BM
    return pl.pallas_call(
        _mid_body,
        grid=(grid,),
        in_specs=[
            pl.BlockSpec((BM, fin), lambda i: (i, 0)),
            pl.BlockSpec((BM, fin), lambda i: (i + NT // BM, 0)),
            pl.BlockSpec((BM, fin), lambda i: (i, 0)),
            pl.BlockSpec((BM, 1), lambda i: (i, 0)),
            pl.BlockSpec((1, fin), lambda i: (0, 0)),
            pl.BlockSpec((fin, fout), lambda i: (0, 0)),
        ],
        out_specs=pl.BlockSpec((BM, fout), lambda i: (i, 0)),
        out_shape=jax.ShapeDtypeStruct((NT, fout), jnp.float32),
    )


def _final_body(p0_ref, p1_ref, g_ref, dinv_ref, b_ref, out_ref):
    a = dinv_ref[...] * (p0_ref[...] + p1_ref[...] + g_ref[...]) + b_ref[...]
    i = pl.program_id(0)
    rid = i * BM + lax.broadcasted_iota(jnp.int32, (BM, 1), 0)
    out_ref[...] = jnp.where(rid < N, a, 0.0)


@functools.cache
def _tc_final(f):
    grid = NT // BM
    return pl.pallas_call(
        _final_body,
        grid=(grid,),
        in_specs=[
            pl.BlockSpec((BM, f), lambda i: (i, 0)),
            pl.BlockSpec((BM, f), lambda i: (i + NT // BM, 0)),
            pl.BlockSpec((BM, f), lambda i: (i, 0)),
            pl.BlockSpec((BM, 1), lambda i: (i, 0)),
            pl.BlockSpec((1, f), lambda i: (0, 0)),
        ],
        out_specs=pl.BlockSpec((BM, f), lambda i: (i, 0)),
        out_shape=jax.ShapeDtypeStruct((NT, f), jnp.float32),
    )


def _match_body(a1_ref, a2_ref, wm1_ref, wm2_ref, f1_ref, f2_ref):
    a1 = a1_ref[...]
    a2 = a2_ref[...]
    m1 = jnp.tanh(jnp.dot(jnp.sum(a1, axis=0, keepdims=True) * (1.0 / N),
                          wm1_ref[...], preferred_element_type=jnp.float32))
    m2 = jnp.tanh(jnp.dot(jnp.sum(a2, axis=0, keepdims=True) * (1.0 / N),
                          wm2_ref[...], preferred_element_type=jnp.float32))
    f1_ref[...] = a1 - m2
    f2_ref[...] = a2 - m1


@functools.cache
def _tc_match(f):
    return pl.pallas_call(
        _match_body,
        out_shape=[
            jax.ShapeDtypeStruct((NT, f), jnp.float32),
            jax.ShapeDtypeStruct((NT, f), jnp.float32),
        ],
    )


def _scores_body(f1_ref, f2_ref, o_ref):
    prod = lax.dot_general(f1_ref[...], f2_ref[...],
                           (((1,), (1,)), ((), ())),
                           preferred_element_type=jnp.float32)
    o_ref[...] = jax.nn.sigmoid(prod)


@functools.cache
def _tc_scores(f):
    bn = 2048
    gm = pl.cdiv(N, BM)
    gn = pl.cdiv(N, bn)
    return pl.pallas_call(
        _scores_body,
        grid=(gm, gn),
        in_specs=[
            pl.BlockSpec((BM, f), lambda i, j: (i, 0)),
            pl.BlockSpec((bn, f), lambda i, j: (j, 0)),
        ],
        out_specs=pl.BlockSpec((BM, bn), lambda i, j: (i, j)),
        out_shape=jax.ShapeDtypeStruct((N, N), jnp.float32),
    )


def _conv_stack(xp, src2d, dst2d, W1, b1, W2, b2, W3, b3, znt, zs):
    hist = _sc_hist_call(dst2d, znt).reshape(NC * NT, 1)
    dinv, g1 = _tc_prep(xp.shape[1], W1.shape[1])(xp, hist, hist, W1)
    p1 = _sc_agg_call(g1, src2d, dst2d, zs[W1.shape[1]])
    g2 = _tc_mid(W1.shape[1], W2.shape[1])(p1, p1, g1, dinv, b1, W2)
    p2 = _sc_agg_call(g2, src2d, dst2d, zs[W2.shape[1]])
    g3 = _tc_mid(W2.shape[1], W3.shape[1])(p2, p2, g2, dinv, b2, W3)
    p3 = _sc_agg_call(g3, src2d, dst2d, zs[W3.shape[1]])
    return _tc_final(W3.shape[1])(p3, p3, g3, dinv, b3)


def kernel(features_1, features_2, edge_index_1, edge_index_2,
           W1, b1, W2, b2, W3, b3, Wm1, Wm2):
    n, d = features_1.shape
    e = edge_index_1.shape[1]
    epad = ((e + NW * CH - 1) // (NW * CH)) * (NW * CH)

    def pad_edges(ei):
        pad = jnp.full((2, epad - e), n, jnp.int32)
        full = jnp.concatenate([ei.astype(jnp.int32), pad], axis=1)
        return full.reshape(2, epad // CH, CH)

    ei1 = pad_edges(edge_index_1)
    ei2 = pad_edges(edge_index_2)
    xp1 = jnp.pad(features_1, ((0, NT - n), (0, 0)))
    xp2 = jnp.pad(features_2, ((0, NT - n), (0, 0)))
    b1r = b1.reshape(1, -1)
    b2r = b2.reshape(1, -1)
    b3r = b3.reshape(1, -1)
    znt = jnp.zeros((NT,), jnp.float32)
    zs = {w.shape[1]: jnp.zeros((NT, w.shape[1]), jnp.float32)
          for w in (W1, W2, W3)}

    a1 = _conv_stack(xp1, ei1[0], ei1[1], W1, b1r, W2, b2r, W3, b3r, znt, zs)
    a2 = _conv_stack(xp2, ei2[0], ei2[1], W1, b1r, W2, b2r, W3, b3r, znt, zs)
    f1e, f2e = _tc_match(W3.shape[1])(a1, a2, Wm1, Wm2)
    return _tc_scores(W3.shape[1])(f1e, f2e)
